# Initial kernel scaffold; baseline (speedup 1.0000x reference)
import jax
import jax.numpy as jnp
from jax.experimental import pallas as pl


def kernel(x_x, x_c, x_b, edge_index_xac, edge_index_cbb, edge_attr_xac, edge_attr_cbb, batch_b, xac1_Wk, xac1_bk, xac1_Wq, xac1_bq, xac1_Wv, xac1_bv, xac1_We, xac1_Wskip, xac1_bias, cbb1_Wk, cbb1_bk, cbb1_Wq, cbb1_bq, cbb1_Wv, cbb1_bv, cbb1_We, cbb1_Wskip, cbb1_bias, cbb2_Wk, cbb2_bk, cbb2_Wq, cbb2_bq, cbb2_Wv, cbb2_bv, cbb2_We, cbb2_Wskip, cbb2_bias, W1, b1, W2, b2, W3, b3, Wo, bo):
    raise NotImplementedError("write your pallas kernel here")


# speedup vs baseline: 1.1486x; 1.1486x over previous
import dataclasses
import functools

import jax
import jax.numpy as jnp
from jax import lax
from jax.experimental import pallas as pl
from jax.experimental.pallas import tpu as pltpu
from jax.experimental.pallas import tpu_sc as plsc

N = 10000
E = 320000
D = 128
G = 64
ROWS = 1000
C = 80
NSUB = 16
FLUSH = 80
NFLUSH = N // FLUSH
MAXF = (NFLUSH + NSUB - 1) // NSUB
HP = jax.lax.Precision.HIGHEST


def _stage_a_body(xx, xc, xb, wqv1, bqv1, wc, bc, wb, bb,
                  qv1, k1, skip1, qv2, k2, skip2):
    y1 = jnp.dot(xx[...], wqv1[...], precision=HP,
                 preferred_element_type=jnp.float32) + bqv1[...]
    qv1[...] = y1
    yc = jnp.dot(xc[...], wc[...], precision=HP,
                 preferred_element_type=jnp.float32) + bc[...]
    k1[...] = yc[:, 0:D]
    skip1[...] = yc[:, D:2 * D]
    qv2[...] = yc[:, 2 * D:4 * D]
    yb = jnp.dot(xb[...], wb[...], precision=HP,
                 preferred_element_type=jnp.float32) + bb[...]
    k2[...] = yb[:, 0:D]
    skip2[...] = yb[:, D:2 * D]


def _stage_a(xx, xc, xb, wqv1, bqv1, wc, bc, wb, bb):
    nblk = N // ROWS
    row_spec = pl.BlockSpec((ROWS, D), lambda i: (i, 0))
    qv_spec = pl.BlockSpec((ROWS, 2 * D), lambda i: (i, 0))
    return pl.pallas_call(
        _stage_a_body,
        grid=(nblk,),
        in_specs=[
            row_spec, row_spec, row_spec,
            pl.BlockSpec((D, 2 * D), lambda i: (0, 0)),
            pl.BlockSpec((1, 2 * D), lambda i: (0, 0)),
            pl.BlockSpec((D, 4 * D), lambda i: (0, 0)),
            pl.BlockSpec((1, 4 * D), lambda i: (0, 0)),
            pl.BlockSpec((D, 2 * D), lambda i: (0, 0)),
            pl.BlockSpec((1, 2 * D), lambda i: (0, 0)),
        ],
        out_specs=[qv_spec, row_spec, row_spec, qv_spec, row_spec, row_spec],
        out_shape=[
            jax.ShapeDtypeStruct((N, 2 * D), jnp.float32),
            jax.ShapeDtypeStruct((N, D), jnp.float32),
            jax.ShapeDtypeStruct((N, D), jnp.float32),
            jax.ShapeDtypeStruct((N, 2 * D), jnp.float32),
            jax.ShapeDtypeStruct((N, D), jnp.float32),
            jax.ShapeDtypeStruct((N, D), jnp.float32),
        ],
    )(xx, xc, xb, wqv1, bqv1, wc, bc, wb, bb)


def _zero_zbuf(zbuf):
    zeros16 = jnp.zeros((16,), jnp.float32)

    @pl.loop(0, FLUSH)
    def _(i):
        for j in range(D // 16):
            zbuf[i, pl.ds(j * 16, 16)] = zeros16


def _zero_acc(acc, zbuf, tile):
    @pl.loop(0, MAXF)
    def _(i):
        c = tile + NSUB * i

        @pl.when(c < NFLUSH)
        def _():
            pltpu.sync_copy(zbuf, acc.at[pl.ds(c * FLUSH, FLUSH)])


def _flush_acc(acc, zbuf, tile, out_ref):
    @pl.loop(0, MAXF)
    def _(i):
        c = tile + NSUB * i

        @pl.when(c < NFLUSH)
        def _():
            pltpu.sync_copy(acc.at[pl.ds(c * FLUSH, FLUSH)], zbuf)
            pltpu.sync_copy(zbuf, out_ref.at[pl.ds(c * FLUSH, FLUSH)])


def _conv_edges(qv_hbm, k_hbm, src_hbm, dst_hbm, ea_hbm, we_hbm,
                edges_per_tile, tile,
                sidx, didx, eab, wev, qvr, krows, msg, acc):
    pltpu.sync_copy(we_hbm, wev)
    base = tile * edges_per_tile
    nchunks = edges_per_tile // C

    @pl.loop(0, nchunks)
    def _(i):
        off = base + i * C
        pltpu.sync_copy(src_hbm.at[pl.ds(off, C)], sidx)
        pltpu.sync_copy(dst_hbm.at[pl.ds(off, C)], didx)
        pltpu.sync_copy(ea_hbm.at[pl.ds(off, C)], eab)
        pltpu.sync_copy(qv_hbm.at[sidx], qvr)
        pltpu.sync_copy(k_hbm.at[didx], krows)

        @pl.loop(0, C)
        def _(e):
            ea_vec = plsc.load_gather(eab, [jnp.full((16,), e, jnp.int32)])
            for j in range(D // 16):
                s = pl.ds(j * 16, 16)
                t = krows[e, s] + qvr[e, s] + ea_vec * wev[s]
                gate = 1.0 / (1.0 + jnp.exp(-t))
                msg[e, s] = gate * qvr[e, pl.ds(D + j * 16, 16)]

        pltpu.sync_copy(msg, acc.at[didx], add=True)


_SC_SCRATCH = [
    pltpu.VMEM((C,), jnp.int32),
    pltpu.VMEM((C,), jnp.int32),
    pltpu.VMEM((C,), jnp.float32),
    pltpu.VMEM((D,), jnp.float32),
    pltpu.VMEM((C, 2 * D), jnp.float32),
    pltpu.VMEM((C, D), jnp.float32),
    pltpu.VMEM((C, D), jnp.float32),
    pltpu.VMEM_SHARED((N, D), jnp.float32),
]

_MESH = plsc.VectorSubcoreMesh(core_axis_name="c", subcore_axis_name="s")

_CP = pltpu.CompilerParams()
if "needs_layout_passes" in pltpu.CompilerParams.__dataclass_fields__:
    _CP = dataclasses.replace(_CP, needs_layout_passes=False)


def _stage_b(qv1, k1, src1, dst1, ea1, we1, qv2, k2, src2, dst2, ea2, we2):
    out_t = [jax.ShapeDtypeStruct((N, D), jnp.float32),
             jax.ShapeDtypeStruct((N, D), jnp.float32)]

    @functools.partial(pl.kernel, out_type=out_t, mesh=_MESH,
                       scratch_types=_SC_SCRATCH, compiler_params=_CP)
    def k(qv1_h, k1_h, s1_h, d1_h, e1_h, w1_h,
          qv2_h, k2_h, s2_h, d2_h, e2_h, w2_h,
          agg1_h, agg2_h,
          sidx, didx, eab, wev, qvr, krows, msg, acc):
        core = lax.axis_index("c")
        tile = lax.axis_index("s")
        _zero_zbuf(msg)
        _zero_acc(acc, msg, tile)
        plsc.subcore_barrier()

        @pl.when(core == 0)
        def _():
            _conv_edges(qv1_h, k1_h, s1_h, d1_h, e1_h, w1_h, E // NSUB, tile,
                        sidx, didx, eab, wev, qvr, krows, msg, acc)

        @pl.when(core == 1)
        def _():
            _conv_edges(qv2_h, k2_h, s2_h, d2_h, e2_h, w2_h, E // NSUB, tile,
                        sidx, didx, eab, wev, qvr, krows, msg, acc)

        plsc.subcore_barrier()

        @pl.when(core == 0)
        def _():
            _flush_acc(acc, msg, tile, agg1_h)

        @pl.when(core == 1)
        def _():
            _flush_acc(acc, msg, tile, agg2_h)

    return k(qv1, k1, src1, dst1, ea1, we1, qv2, k2, src2, dst2, ea2, we2)


def _stage_d(qv3, k3, src, dst, ea, we):
    out_t = [jax.ShapeDtypeStruct((2, N, D), jnp.float32)]

    @functools.partial(pl.kernel, out_type=out_t, mesh=_MESH,
                       scratch_types=_SC_SCRATCH, compiler_params=_CP)
    def k(qv_h, k_h, s_h, d_h, e_h, w_h, part_h,
          sidx, didx, eab, wev, qvr, krows, msg, acc):
        core = lax.axis_index("c")
        tile = lax.axis_index("s")
        _zero_zbuf(msg)
        _zero_acc(acc, msg, tile)
        plsc.subcore_barrier()
        wid = core * NSUB + tile
        per_worker = E // (2 * NSUB)
        pltpu.sync_copy(w_h, wev)
        base = wid * per_worker
        nchunks = per_worker // C

        @pl.loop(0, nchunks)
        def _(i):
            off = base + i * C
            pltpu.sync_copy(s_h.at[pl.ds(off, C)], sidx)
            pltpu.sync_copy(d_h.at[pl.ds(off, C)], didx)
            pltpu.sync_copy(e_h.at[pl.ds(off, C)], eab)
            pltpu.sync_copy(qv_h.at[sidx], qvr)
            pltpu.sync_copy(k_h.at[didx], krows)

            @pl.loop(0, C)
            def _(e):
                ea_vec = plsc.load_gather(eab, [jnp.full((16,), e, jnp.int32)])
                for j in range(D // 16):
                    s = pl.ds(j * 16, 16)
                    t = krows[e, s] + qvr[e, s] + ea_vec * wev[s]
                    gate = 1.0 / (1.0 + jnp.exp(-t))
                    msg[e, s] = gate * qvr[e, pl.ds(D + j * 16, 16)]

            pltpu.sync_copy(msg, acc.at[didx], add=True)

        plsc.subcore_barrier()

        @pl.loop(0, MAXF)
        def _(i):
            c = tile + NSUB * i

            @pl.when(c < NFLUSH)
            def _():
                pltpu.sync_copy(acc.at[pl.ds(c * FLUSH, FLUSH)], msg)
                pltpu.sync_copy(msg, part_h.at[core, pl.ds(c * FLUSH, FLUSH)])

    return k(qv3, k3, src, dst, ea, we)[0]


def _stage_c_body(agg1, agg2, skip1, skip2, wqv3, bqv3, wks3, bks3,
                  qv3, k3, skip3):
    c1 = jnp.maximum(agg1[...] + skip1[...], 0.0)
    b1 = jnp.maximum(agg2[...] + skip2[...], 0.0)
    qv3[...] = jnp.dot(c1, wqv3[...], precision=HP,
                       preferred_element_type=jnp.float32) + bqv3[...]
    yk = jnp.dot(b1, wks3[...], precision=HP,
                 preferred_element_type=jnp.float32) + bks3[...]
    k3[...] = yk[:, 0:D]
    skip3[...] = yk[:, D:2 * D]


def _stage_c(agg1, agg2, skip1, skip2, wqv3, bqv3, wks3, bks3):
    nblk = N // ROWS
    row_spec = pl.BlockSpec((ROWS, D), lambda i: (i, 0))
    qv_spec = pl.BlockSpec((ROWS, 2 * D), lambda i: (i, 0))
    return pl.pallas_call(
        _stage_c_body,
        grid=(nblk,),
        in_specs=[
            row_spec, row_spec, row_spec, row_spec,
            pl.BlockSpec((D, 2 * D), lambda i: (0, 0)),
            pl.BlockSpec((1, 2 * D), lambda i: (0, 0)),
            pl.BlockSpec((D, 2 * D), lambda i: (0, 0)),
            pl.BlockSpec((1, 2 * D), lambda i: (0, 0)),
        ],
        out_specs=[qv_spec, row_spec, row_spec],
        out_shape=[
            jax.ShapeDtypeStruct((N, 2 * D), jnp.float32),
            jax.ShapeDtypeStruct((N, D), jnp.float32),
            jax.ShapeDtypeStruct((N, D), jnp.float32),
        ],
    )(agg1, agg2, skip1, skip2, wqv3, bqv3, wks3, bks3)


def _stage_e_body(p0, p1, skip3, batch, w1, b1, w2, b2, w3, b3, wo, bo,
                  out, sums, cnts):
    step = pl.program_id(0)
    nsteps = pl.num_programs(0)

    @pl.when(step == 0)
    def _():
        sums[...] = jnp.zeros_like(sums)
        cnts[...] = jnp.zeros_like(cnts)

    b2v = jnp.maximum(p0[...] + p1[...] + skip3[...], 0.0)
    iota = lax.broadcasted_iota(jnp.int32, (ROWS, G), 1)
    oh = (batch[...] == iota).astype(jnp.float32)
    dn = (((0,), (0,)), ((), ()))
    sums[...] += lax.dot_general(oh, b2v, dn, precision=HP,
                                 preferred_element_type=jnp.float32)
    cnts[...] += lax.dot_general(oh, jnp.ones((ROWS, D), jnp.float32), dn,
                                 precision=HP,
                                 preferred_element_type=jnp.float32)

    @pl.when(step == nsteps - 1)
    def _():
        h = sums[...] / jnp.maximum(cnts[...], 1.0)
        h = jnp.maximum(jnp.dot(h, w1[...], precision=HP,
                                preferred_element_type=jnp.float32) + b1[...], 0.0)
        h = jnp.maximum(jnp.dot(h, w2[...], precision=HP,
                                preferred_element_type=jnp.float32) + b2[...], 0.0)
        h = jnp.maximum(jnp.dot(h, w3[...], precision=HP,
                                preferred_element_type=jnp.float32) + b3[...], 0.0)
        out[...] = jnp.dot(h, wo[...], precision=HP,
                           preferred_element_type=jnp.float32) + bo[...]


def _stage_e(p0, p1, skip3, batch2d, w1, b1, w2, b2, w3, b3, wo, bo):
    nblk = N // ROWS
    row_spec = pl.BlockSpec((ROWS, D), lambda i: (i, 0))
    full = pl.BlockSpec((D, D), lambda i: (0, 0))
    vec = pl.BlockSpec((1, D), lambda i: (0, 0))
    return pl.pallas_call(
        _stage_e_body,
        grid=(nblk,),
        in_specs=[
            row_spec, row_spec, row_spec,
            pl.BlockSpec((ROWS, 1), lambda i: (i, 0)),
            full, vec, full, vec, full, vec,
            pl.BlockSpec((D, 1), lambda i: (0, 0)),
            pl.BlockSpec((1, 1), lambda i: (0, 0)),
        ],
        out_specs=[pl.BlockSpec((G, 1), lambda i: (0, 0))],
        out_shape=[jax.ShapeDtypeStruct((G, 1), jnp.float32)],
        scratch_shapes=[pltpu.VMEM((G, D), jnp.float32),
                        pltpu.VMEM((G, D), jnp.float32)],
    )(p0, p1, skip3, batch2d, w1, b1, w2, b2, w3, b3, wo, bo)[0]


def kernel(x_x, x_c, x_b, edge_index_xac, edge_index_cbb, edge_attr_xac,
           edge_attr_cbb, batch_b,
           xac1_Wk, xac1_bk, xac1_Wq, xac1_bq, xac1_Wv, xac1_bv, xac1_We,
           xac1_Wskip, xac1_bias,
           cbb1_Wk, cbb1_bk, cbb1_Wq, cbb1_bq, cbb1_Wv, cbb1_bv, cbb1_We,
           cbb1_Wskip, cbb1_bias,
           cbb2_Wk, cbb2_bk, cbb2_Wq, cbb2_bq, cbb2_Wv, cbb2_bv, cbb2_We,
           cbb2_Wskip, cbb2_bias,
           W1, b1, W2, b2, W3, b3, Wo, bo):
    f32 = jnp.float32
    wqv1 = jnp.concatenate([xac1_Wq, xac1_Wv], axis=1)
    bqv1 = jnp.concatenate([xac1_bq, xac1_bv]).reshape(1, 2 * D)
    wc = jnp.concatenate([xac1_Wk, xac1_Wskip, cbb1_Wq, cbb1_Wv], axis=1)
    bc = jnp.concatenate([xac1_bk, xac1_bias, cbb1_bq, cbb1_bv]).reshape(1, 4 * D)
    wb = jnp.concatenate([cbb1_Wk, cbb1_Wskip], axis=1)
    bb = jnp.concatenate([cbb1_bk, cbb1_bias]).reshape(1, 2 * D)
    wqv3 = jnp.concatenate([cbb2_Wq, cbb2_Wv], axis=1)
    bqv3 = jnp.concatenate([cbb2_bq, cbb2_bv]).reshape(1, 2 * D)
    wks3 = jnp.concatenate([cbb2_Wk, cbb2_Wskip], axis=1)
    bks3 = jnp.concatenate([cbb2_bk, cbb2_bias]).reshape(1, 2 * D)

    src1 = edge_index_xac[0].astype(jnp.int32)
    dst1 = edge_index_xac[1].astype(jnp.int32)
    src2 = edge_index_cbb[0].astype(jnp.int32)
    dst2 = edge_index_cbb[1].astype(jnp.int32)
    ea1 = edge_attr_xac.reshape(E).astype(f32)
    ea2 = edge_attr_cbb.reshape(E).astype(f32)
    we1 = xac1_We.reshape(D).astype(f32)
    we2 = cbb1_We.reshape(D).astype(f32)
    we3 = cbb2_We.reshape(D).astype(f32)

    qv1, k1, skip1, qv2, k2, skip2 = _stage_a(
        x_x, x_c, x_b, wqv1, bqv1, wc, bc, wb, bb)

    agg1, agg2 = _stage_b(qv1, k1, src1, dst1, ea1, we1,
                          qv2, k2, src2, dst2, ea2, we2)

    qv3, k3, skip3 = _stage_c(agg1, agg2, skip1, skip2,
                              wqv3, bqv3, wks3, bks3)

    part = _stage_d(qv3, k3, src2, dst2, ea2, we3)

    out = _stage_e(part[0], part[1], skip3, batch_b.reshape(N, 1).astype(jnp.int32),
                   W1, b1.reshape(1, D), W2, b2.reshape(1, D),
                   W3, b3.reshape(1, D), Wo, bo.reshape(1, 1))
    return out.reshape(-1)
